# baseline (speedup 1.0000x reference)
import functools

import jax
import jax.numpy as jnp
from jax import lax
from jax.experimental import pallas as pl
from jax.experimental.pallas import tpu as pltpu

_LN_EPS = 1e-5


def _ln_block(z, g, beta, out_dtype):
    d = z.shape[-1]
    inv_d = jnp.float32(1.0 / d)
    mean = jnp.sum(z, axis=-1, keepdims=True) * inv_d
    ex2 = jnp.sum(z * z, axis=-1, keepdims=True) * inv_d
    var = jnp.maximum(ex2 - mean * mean, 0.0)
    rstd = lax.rsqrt(var + _LN_EPS)
    scale = rstd * g
    shift = beta - mean * scale
    return (z * scale + shift).astype(out_dtype)


def _pipe_kernel(x_hbm, wt_ref, b_ref, g_ref, beta_ref, o_hbm,
                 x_buf, z_buf, o_buf, in_sems, out_sems,
                 *, tm, n_steps, n_in, n_out):
    def dma_in(slot, step):
        pltpu.make_async_copy(
            x_hbm.at[pl.ds(step * tm, tm), :], x_buf.at[slot],
            in_sems.at[slot]).start()

    def wait_in(slot):
        pltpu.make_async_copy(
            x_buf.at[slot], x_buf.at[slot], in_sems.at[slot]).wait()

    def dma_out(slot, step):
        pltpu.make_async_copy(
            o_buf.at[slot], o_hbm.at[pl.ds(step * tm, tm), :],
            out_sems.at[slot]).start()

    def wait_out(slot):
        pltpu.make_async_copy(
            o_buf.at[slot], o_buf.at[slot], out_sems.at[slot]).wait()

    for i in range(min(n_in, n_steps)):
        dma_in(i, i)

    b = b_ref[...]
    g = g_ref[...]
    beta = beta_ref[...]

    def compute_z(step_slot, zslot):
        x = x_buf[step_slot]
        y = jnp.dot(x.astype(jnp.bfloat16), wt_ref[...],
                    preferred_element_type=jnp.float32)
        z_buf[zslot] = y + x + b

    wait_in(0)
    compute_z(0, 0)
    if n_in < n_steps:
        dma_in(0, n_in)

    def body(step, carry):
        cur = lax.rem(step, n_in)
        wait_in(cur)
        compute_z(cur, lax.rem(step, 2))

        prev = step - 1
        oprev = lax.rem(prev, n_out)

        @pl.when(prev >= n_out)
        def _():
            wait_out(oprev)

        o_buf[oprev] = _ln_block(z_buf[lax.rem(prev, 2)], g, beta,
                                 o_buf.dtype)
        dma_out(oprev, prev)

        @pl.when(step + n_in < n_steps)
        def _():
            dma_in(cur, step + n_in)

        return carry

    lax.fori_loop(1, n_steps, body, 0)

    last = n_steps - 1
    olast = lax.rem(last, n_out)
    if last >= n_out:
        wait_out(olast)
    o_buf[olast] = _ln_block(z_buf[lax.rem(last, 2)], g, beta, o_buf.dtype)
    dma_out(olast, last)
    for i in range(min(n_out, n_steps)):
        wait_out(i)


@functools.partial(jax.jit, static_argnames=("tm", "n_in", "n_out"))
def forward_v5(x, w, b, gamma, beta, *, tm=512, n_in=4, n_out=4):
    B, S, D = x.shape
    R = B * S
    TM = min(tm, R)
    n_steps = pl.cdiv(R, TM)
    R_pad = n_steps * TM

    x2 = x.reshape(R, D)
    if R_pad != R:
        x2 = jnp.pad(x2, ((0, R_pad - R), (0, 0)))
    wt = jnp.asarray(w).T.astype(jnp.bfloat16)
    b2 = b.reshape(1, D).astype(jnp.float32)
    g2 = gamma.reshape(1, D).astype(jnp.float32)
    beta2 = beta.reshape(1, D).astype(jnp.float32)

    kernel_fn = functools.partial(
        _pipe_kernel, tm=TM, n_steps=n_steps, n_in=n_in, n_out=n_out)
    out2 = pl.pallas_call(
        kernel_fn,
        out_shape=jax.ShapeDtypeStruct((R_pad, D), x.dtype),
        in_specs=[
            pl.BlockSpec(memory_space=pl.ANY),
            pl.BlockSpec(memory_space=pltpu.VMEM),
            pl.BlockSpec(memory_space=pltpu.VMEM),
            pl.BlockSpec(memory_space=pltpu.VMEM),
            pl.BlockSpec(memory_space=pltpu.VMEM),
        ],
        out_specs=pl.BlockSpec(memory_space=pl.ANY),
        scratch_shapes=[
            pltpu.VMEM((n_in, TM, D), x.dtype),
            pltpu.VMEM((2, TM, D), jnp.float32),
            pltpu.VMEM((n_out, TM, D), x.dtype),
            pltpu.SemaphoreType.DMA((n_in,)),
            pltpu.SemaphoreType.DMA((n_out,)),
        ],
        compiler_params=pltpu.CompilerParams(
            vmem_limit_bytes=56 * 1024 * 1024,
        ),
    )(x2, wt, b2, g2, beta2)
    return out2[:R].reshape(B, S, D)


def kernel(x, w, b, gamma, beta):
    return forward_v5(x, w, b, gamma, beta, tm=512, n_in=4, n_out=4)


if __name__ == "__main__":
    import numpy as np
    from jax.experimental.pallas import tpu as pltpu

    key = jax.random.key(0)
    kx, kw, kb, kg, kbt = jax.random.split(key, 5)
    B, S, D = 2, 16, 256
    x = jax.random.normal(kx, (B, S, D), dtype=jnp.float32)
    w = jax.random.uniform(kw, (D, D), minval=-0.03, maxval=0.03,
                           dtype=jnp.float32)
    b = jax.random.uniform(kb, (D,), minval=-0.03, maxval=0.03,
                           dtype=jnp.float32)
    g = 1.0 + 0.1 * jax.random.normal(kg, (D,), dtype=jnp.float32)
    bt = 0.02 * jax.random.normal(kbt, (D,), dtype=jnp.float32)

    with pltpu.force_tpu_interpret_mode():
        out = forward_v5(x, w, b, g, bt, tm=8, n_in=3, n_out=2)

    y = x @ w.T + b
    z = y + x
    mu = z.mean(-1, keepdims=True)
    v = z.var(-1, keepdims=True)
    ref = (z - mu) * jax.lax.rsqrt(v + 1e-5) * g + bt
    print("max_abs_err", jnp.abs(out - ref).max(),
          "resid_var_ratio", jnp.var(out - ref) / jnp.var(ref))


# speedup vs baseline: 1.0596x; 1.0596x over previous
import functools

import jax
import jax.numpy as jnp
from jax import lax
from jax.experimental import pallas as pl
from jax.experimental.pallas import tpu as pltpu

_LN_EPS = 1e-5


def _pipe_kernel(x_hbm, wt_ref, b_ref, g_ref, beta_ref, o_hbm,
                 x_buf, z_a, z_b, o_buf, in_sems, out_sems,
                 *, tm, n_steps, n_in, n_out):
    assert n_steps % 2 == 0 and n_steps >= max(n_in, n_out) + 4

    def dma_in(slot, step):
        pltpu.make_async_copy(
            x_hbm.at[pl.ds(step * tm, tm), :], x_buf.at[slot],
            in_sems.at[slot]).start()

    def wait_in(slot):
        pltpu.make_async_copy(
            x_buf.at[slot], x_buf.at[slot], in_sems.at[slot]).wait()

    def dma_out(slot, step):
        pltpu.make_async_copy(
            o_buf.at[slot], o_hbm.at[pl.ds(step * tm, tm), :],
            out_sems.at[slot]).start()

    def wait_out(slot):
        pltpu.make_async_copy(
            o_buf.at[slot], o_buf.at[slot], out_sems.at[slot]).wait()

    b = b_ref[...]
    g = g_ref[...]
    beta = beta_ref[...]
    inv_d = jnp.float32(1.0 / wt_ref.shape[1])

    def dot_into(z_ref, tile):
        slot = lax.rem(tile, n_in)
        wait_in(slot)
        x = x_buf[slot]
        y = jnp.dot(x.astype(jnp.bfloat16), wt_ref[...],
                    preferred_element_type=jnp.float32)
        z_ref[...] = y + x + b

    def ln_store(z_ref, tile, *, do_wait):
        z = z_ref[...]
        mean = jnp.sum(z, axis=-1, keepdims=True) * inv_d
        ex2 = jnp.sum(z * z, axis=-1, keepdims=True) * inv_d
        var = jnp.maximum(ex2 - mean * mean, 0.0)
        rstd = lax.rsqrt(var + _LN_EPS)
        out = ((z - mean) * rstd) * g + beta
        oslot = lax.rem(tile, n_out)
        if do_wait:
            wait_out(oslot)
        o_buf[oslot] = out.astype(o_buf.dtype)
        dma_out(oslot, tile)

    for i in range(n_in):
        dma_in(i, i)

    dot_into(z_a, 0)
    dma_in(0, n_in)

    for t in (1, 2):
        dot_into(z_b, 2 * t - 1)
        ln_store(z_a, 2 * t - 2, do_wait=(2 * t - 2 >= n_out))
        dot_into(z_a, 2 * t)
        ln_store(z_b, 2 * t - 1, do_wait=(2 * t - 1 >= n_out))
        for m in (2 * t + n_in - 1, 2 * t + n_in):
            if m < n_steps:
                dma_in(m % n_in, m)

    def body(t, carry):
        t1 = 2 * t - 1
        t2 = 2 * t
        dot_into(z_b, t1)
        ln_store(z_a, t1 - 1, do_wait=True)
        dot_into(z_a, t2)
        ln_store(z_b, t1, do_wait=True)

        @pl.when(t2 + n_in - 1 < n_steps)
        def _():
            dma_in(lax.rem(t1, n_in), t1 + n_in)

        @pl.when(t2 + n_in < n_steps)
        def _():
            dma_in(lax.rem(t2, n_in), t2 + n_in)

        return carry

    lax.fori_loop(3, n_steps // 2, body, 0)

    last = n_steps - 1
    dot_into(z_b, last)
    ln_store(z_a, last - 1, do_wait=True)
    ln_store(z_b, last, do_wait=True)
    for i in range(n_out):
        wait_out(i)


@functools.partial(jax.jit, static_argnames=("tm", "n_in", "n_out"))
def forward_v6(x, w, b, gamma, beta, *, tm=512, n_in=6, n_out=4):
    B, S, D = x.shape
    R = B * S
    TM = min(tm, R)
    n_steps = pl.cdiv(R, TM)
    R_pad = n_steps * TM

    x2 = x.reshape(R, D)
    if R_pad != R:
        x2 = jnp.pad(x2, ((0, R_pad - R), (0, 0)))
    wt = jnp.asarray(w).T.astype(jnp.bfloat16)
    b2 = b.reshape(1, D).astype(jnp.float32)
    g2 = gamma.reshape(1, D).astype(jnp.float32)
    beta2 = beta.reshape(1, D).astype(jnp.float32)

    kernel_fn = functools.partial(
        _pipe_kernel, tm=TM, n_steps=n_steps, n_in=n_in, n_out=n_out)
    out2 = pl.pallas_call(
        kernel_fn,
        out_shape=jax.ShapeDtypeStruct((R_pad, D), x.dtype),
        in_specs=[
            pl.BlockSpec(memory_space=pl.ANY),
            pl.BlockSpec(memory_space=pltpu.VMEM),
            pl.BlockSpec(memory_space=pltpu.VMEM),
            pl.BlockSpec(memory_space=pltpu.VMEM),
            pl.BlockSpec(memory_space=pltpu.VMEM),
        ],
        out_specs=pl.BlockSpec(memory_space=pl.ANY),
        scratch_shapes=[
            pltpu.VMEM((n_in, TM, D), x.dtype),
            pltpu.VMEM((TM, D), jnp.float32),
            pltpu.VMEM((TM, D), jnp.float32),
            pltpu.VMEM((n_out, TM, D), x.dtype),
            pltpu.SemaphoreType.DMA((n_in,)),
            pltpu.SemaphoreType.DMA((n_out,)),
        ],
        compiler_params=pltpu.CompilerParams(
            vmem_limit_bytes=56 * 1024 * 1024,
        ),
    )(x2, wt, b2, g2, beta2)
    return out2[:R].reshape(B, S, D)


def kernel(x, w, b, gamma, beta):
    return forward_v6(x, w, b, gamma, beta, tm=512, n_in=6, n_out=4)


if __name__ == "__main__":
    key = jax.random.key(0)
    kx, kw, kb, kg, kbt = jax.random.split(key, 5)
    B, S, D = 2, 48, 256
    x = jax.random.normal(kx, (B, S, D), dtype=jnp.float32)
    w = jax.random.uniform(kw, (D, D), minval=-0.03, maxval=0.03,
                           dtype=jnp.float32)
    b = jax.random.uniform(kb, (D,), minval=-0.03, maxval=0.03,
                           dtype=jnp.float32)
    g = 1.0 + 0.1 * jax.random.normal(kg, (D,), dtype=jnp.float32)
    bt = 0.02 * jax.random.normal(kbt, (D,), dtype=jnp.float32)

    with pltpu.force_tpu_interpret_mode():
        out = forward_v6(x, w, b, g, bt, tm=8, n_in=4, n_out=3)

    y = x @ w.T + b
    z = y + x
    mu = z.mean(-1, keepdims=True)
    v = z.var(-1, keepdims=True)
    ref = (z - mu) * jax.lax.rsqrt(v + 1e-5) * g + bt
    print("max_abs_err", jnp.abs(out - ref).max(),
          "resid_var_ratio", jnp.var(out - ref) / jnp.var(ref))
